# baseline (speedup 1.0000x reference)
import jax
import jax.numpy as jnp
from jax import lax
from jax.experimental import pallas as pl
from jax.experimental.pallas import tpu as pltpu
from jax.experimental.pallas import tpu_sc as plsc

N = 10000
NP = 10240
E = 320000
IN_CH = 128
HID = 64

NC = 2
NS = 16
NW = NC * NS
K = 128
NBUF = 6
CH = 84
EP = NW * K * CH
RPT = NP // NS

RB = 1024
NB = NP // RB


def _sc_mesh():
    return plsc.VectorSubcoreMesh(
        core_axis_name="c", subcore_axis_name="s", num_cores=NC, num_subcores=NS
    )


def _sc_agg(ec, y, d):
    def body(src_h, dst_h, y_h, zz_h, out_h, sall, dall, rw0, rw1, acc, sg0, sg1):
        rows = [rw0, rw1]
        sems = [sg0, sg1]
        c = lax.axis_index("c")
        s = lax.axis_index("s")
        wid = c * NS + s
        r0 = s * RPT
        base = wid * CH
        pltpu.sync_copy(src_h.at[pl.ds(base, CH)], sall)
        pltpu.sync_copy(dst_h.at[pl.ds(base, CH)], dall)
        pltpu.sync_copy(zz_h.at[pl.ds(r0, RPT)], acc.at[pl.ds(r0, RPT)])
        plsc.subcore_barrier()

        def gather_start(t, b):
            pltpu.async_copy(y_h.at[sall.at[t]], rows[b], sems[b])

        def finish(t, b):
            pltpu.make_async_copy(y_h.at[sall.at[t]], rows[b], sems[b]).wait()
            pltpu.sync_copy(rows[b], acc.at[dall.at[t]], add=True)

        def step(t, carry):
            gather_start(t, 0)
            finish(t, 0)
            return carry

        lax.fori_loop(0, CH, step, 0)

        plsc.subcore_barrier()
        pltpu.sync_copy(acc.at[pl.ds(r0, RPT)], out_h.at[c, pl.ds(r0, RPT)])

    zeros = jnp.zeros((NP, d), jnp.float32)
    f = pl.kernel(
        body,
        out_type=jax.ShapeDtypeStruct((NC, NP, d), jnp.float32),
        mesh=_sc_mesh(),
        compiler_params=pltpu.CompilerParams(use_tc_tiling_on_sc=False),
        scratch_types=[
            pltpu.VMEM((CH, K), jnp.int32),
            pltpu.VMEM((CH, K), jnp.int32),
            pltpu.VMEM((K, d), jnp.float32),
            pltpu.VMEM((K, d), jnp.float32),
            pltpu.VMEM_SHARED((NP, d), jnp.float32),
            pltpu.SemaphoreType.DMA,
            pltpu.SemaphoreType.DMA,
        ],
    )
    return f(ec[0], ec[1], y, zeros)


def _tc_first(d0, d1, x, W1):
    def body(d0_r, d1_r, x_r, w_r, dis_r, y_r):
        dis = lax.rsqrt(d0_r[...] + d1_r[...] + 1.0)
        xw = jnp.dot(x_r[...], w_r[...], preferred_element_type=jnp.float32)
        dis_r[...] = dis
        y_r[...] = dis * xw

    return pl.pallas_call(
        body,
        grid=(NB,),
        in_specs=[
            pl.BlockSpec((RB, 1), lambda i: (i, 0)),
            pl.BlockSpec((RB, 1), lambda i: (i, 0)),
            pl.BlockSpec((RB, IN_CH), lambda i: (i, 0)),
            pl.BlockSpec((IN_CH, HID), lambda i: (0, 0)),
        ],
        out_specs=[
            pl.BlockSpec((RB, 1), lambda i: (i, 0)),
            pl.BlockSpec((RB, HID), lambda i: (i, 0)),
        ],
        out_shape=[
            jax.ShapeDtypeStruct((NP, 1), jnp.float32),
            jax.ShapeDtypeStruct((NP, HID), jnp.float32),
        ],
    )(d0, d1, x, W1)


def _tc_mid(p0, p1, y, dis, b, W, d_out):
    def body(p0_r, p1_r, y_r, dis_r, b_r, w_r, o_r):
        dis = dis_r[...]
        h = jnp.maximum(dis * (p0_r[...] + p1_r[...] + y_r[...]) + b_r[...], 0.0)
        o_r[...] = dis * jnp.dot(h, w_r[...], preferred_element_type=jnp.float32)

    return pl.pallas_call(
        body,
        grid=(NB,),
        in_specs=[
            pl.BlockSpec((RB, HID), lambda i: (i, 0)),
            pl.BlockSpec((RB, HID), lambda i: (i, 0)),
            pl.BlockSpec((RB, HID), lambda i: (i, 0)),
            pl.BlockSpec((RB, 1), lambda i: (i, 0)),
            pl.BlockSpec((1, HID), lambda i: (0, 0)),
            pl.BlockSpec((HID, d_out), lambda i: (0, 0)),
        ],
        out_specs=pl.BlockSpec((RB, d_out), lambda i: (i, 0)),
        out_shape=jax.ShapeDtypeStruct((NP, d_out), jnp.float32),
    )(p0, p1, y, dis, b, W)


def _tc_last(r0, r1, z, dis, b3):
    def body(r0_r, r1_r, z_r, dis_r, b_r, o_r):
        o_r[...] = dis_r[...] * (r0_r[...] + r1_r[...] + z_r[...]) + b_r[...]

    return pl.pallas_call(
        body,
        grid=(NB,),
        in_specs=[
            pl.BlockSpec((RB, 1), lambda i: (i, 0)),
            pl.BlockSpec((RB, 1), lambda i: (i, 0)),
            pl.BlockSpec((RB, 1), lambda i: (i, 0)),
            pl.BlockSpec((RB, 1), lambda i: (i, 0)),
            pl.BlockSpec((1, 1), lambda i: (0, 0)),
        ],
        out_specs=pl.BlockSpec((RB, 1), lambda i: (i, 0)),
        out_shape=jax.ShapeDtypeStruct((NP, 1), jnp.float32),
    )(r0, r1, z, dis, b3)


def kernel(x, edge_index, W1, b1, W2, b2, W3, b3):
    ei = edge_index.astype(jnp.int32)
    pad = jnp.full((2, EP - E), N, jnp.int32)
    ec = jnp.concatenate([ei, pad], axis=1).reshape(2, NW * CH, K)
    xp = jnp.zeros((NP, IN_CH), jnp.float32).at[:N].set(x)

    ones1 = jnp.ones((NP, 1), jnp.float32)
    degp = _sc_agg(ec, ones1, 1)
    dis, y1 = _tc_first(degp[0], degp[1], xp, W1)

    p = _sc_agg(ec, y1, HID)
    y2 = _tc_mid(p[0], p[1], y1, dis, b1.reshape(1, HID), W2, HID)

    q = _sc_agg(ec, y2, HID)
    z = _tc_mid(q[0], q[1], y2, dis, b2.reshape(1, HID), W3, 1)

    r = _sc_agg(ec, z, 1)
    out = _tc_last(r[0], r[1], z, dis, b3.reshape(1, 1))
    return out[:N, 0]


# speedup vs baseline: 1.8175x; 1.8175x over previous
import jax
import jax.numpy as jnp
from jax import lax
from jax.experimental import pallas as pl
from jax.experimental.pallas import tpu as pltpu
from jax.experimental.pallas import tpu_sc as plsc

N = 10000
NP = 10240
E = 320000
IN_CH = 128
HID = 64

NC = 2
NS = 16
NW = NC * NS
K = 128
NBUF = 6
CH = 79
EP = NW * K * CH
RPT = NP // NS

RB = 1024
NB = NP // RB


def _sc_mesh():
    return plsc.VectorSubcoreMesh(
        core_axis_name="c", subcore_axis_name="s", num_cores=NC, num_subcores=NS
    )


def _sc_agg(ec, y, d):
    def body(src_h, dst_h, y_h, zz_h, out_h, sidx, didx, rows, acc, sem):
        c = lax.axis_index("c")
        s = lax.axis_index("s")
        wid = c * NS + s
        r0 = s * RPT
        pltpu.sync_copy(zz_h.at[pl.ds(r0, RPT)], acc.at[pl.ds(r0, RPT)])
        plsc.subcore_barrier()
        base = wid * (K * CH)

        def step(i, carry):
            off = base + i * K
            pltpu.sync_copy(src_h.at[pl.ds(off, K)], sidx)
            pltpu.sync_copy(dst_h.at[pl.ds(off, K)], didx)
            pltpu.async_copy(y_h.at[sidx], rows, sem).wait()
            pltpu.sync_copy(rows, acc.at[didx], add=True)
            return carry

        lax.fori_loop(0, CH, step, 0)
        plsc.subcore_barrier()
        pltpu.sync_copy(acc.at[pl.ds(r0, RPT)], out_h.at[c, pl.ds(r0, RPT)])

    zeros = jnp.zeros((NP, d), jnp.float32)
    f = pl.kernel(
        body,
        out_type=jax.ShapeDtypeStruct((NC, NP, d), jnp.float32),
        mesh=_sc_mesh(),
        compiler_params=pltpu.CompilerParams(use_tc_tiling_on_sc=False),
        scratch_types=[
            pltpu.VMEM((K,), jnp.int32),
            pltpu.VMEM((K,), jnp.int32),
            pltpu.VMEM((K, d), jnp.float32),
            pltpu.VMEM_SHARED((NP, d), jnp.float32),
            pltpu.SemaphoreType.DMA,
        ],
    )
    return f(ec[0], ec[1], y, zeros)


def _tc_first(d0, d1, x, W1):
    def body(d0_r, d1_r, x_r, w_r, dis_r, y_r):
        dis = lax.rsqrt(d0_r[...] + d1_r[...] + 1.0)
        xw = jnp.dot(x_r[...], w_r[...], preferred_element_type=jnp.float32)
        dis_r[...] = dis
        y_r[...] = dis * xw

    return pl.pallas_call(
        body,
        grid=(NB,),
        in_specs=[
            pl.BlockSpec((RB, 1), lambda i: (i, 0)),
            pl.BlockSpec((RB, 1), lambda i: (i, 0)),
            pl.BlockSpec((RB, IN_CH), lambda i: (i, 0)),
            pl.BlockSpec((IN_CH, HID), lambda i: (0, 0)),
        ],
        out_specs=[
            pl.BlockSpec((RB, 1), lambda i: (i, 0)),
            pl.BlockSpec((RB, HID), lambda i: (i, 0)),
        ],
        out_shape=[
            jax.ShapeDtypeStruct((NP, 1), jnp.float32),
            jax.ShapeDtypeStruct((NP, HID), jnp.float32),
        ],
    )(d0, d1, x, W1)


def _tc_mid(p0, p1, y, dis, b, W, d_out):
    def body(p0_r, p1_r, y_r, dis_r, b_r, w_r, o_r):
        dis = dis_r[...]
        h = jnp.maximum(dis * (p0_r[...] + p1_r[...] + y_r[...]) + b_r[...], 0.0)
        o_r[...] = dis * jnp.dot(h, w_r[...], preferred_element_type=jnp.float32)

    return pl.pallas_call(
        body,
        grid=(NB,),
        in_specs=[
            pl.BlockSpec((RB, HID), lambda i: (i, 0)),
            pl.BlockSpec((RB, HID), lambda i: (i, 0)),
            pl.BlockSpec((RB, HID), lambda i: (i, 0)),
            pl.BlockSpec((RB, 1), lambda i: (i, 0)),
            pl.BlockSpec((1, HID), lambda i: (0, 0)),
            pl.BlockSpec((HID, d_out), lambda i: (0, 0)),
        ],
        out_specs=pl.BlockSpec((RB, d_out), lambda i: (i, 0)),
        out_shape=jax.ShapeDtypeStruct((NP, d_out), jnp.float32),
    )(p0, p1, y, dis, b, W)


def _tc_last(r0, r1, z, dis, b3):
    def body(r0_r, r1_r, z_r, dis_r, b_r, o_r):
        o_r[...] = dis_r[...] * (r0_r[...] + r1_r[...] + z_r[...]) + b_r[...]

    return pl.pallas_call(
        body,
        grid=(NB,),
        in_specs=[
            pl.BlockSpec((RB, 1), lambda i: (i, 0)),
            pl.BlockSpec((RB, 1), lambda i: (i, 0)),
            pl.BlockSpec((RB, 1), lambda i: (i, 0)),
            pl.BlockSpec((RB, 1), lambda i: (i, 0)),
            pl.BlockSpec((1, 1), lambda i: (0, 0)),
        ],
        out_specs=pl.BlockSpec((RB, 1), lambda i: (i, 0)),
        out_shape=jax.ShapeDtypeStruct((NP, 1), jnp.float32),
    )(r0, r1, z, dis, b3)


def kernel(x, edge_index, W1, b1, W2, b2, W3, b3):
    ei = edge_index.astype(jnp.int32)
    pad = jnp.full((2, EP - E), N, jnp.int32)
    ec = jnp.concatenate([ei, pad], axis=1)
    xp = jnp.zeros((NP, IN_CH), jnp.float32).at[:N].set(x)

    ones1 = jnp.ones((NP, 1), jnp.float32)
    degp = _sc_agg(ec, ones1, 1)
    dis, y1 = _tc_first(degp[0], degp[1], xp, W1)

    p = _sc_agg(ec, y1, HID)
    y2 = _tc_mid(p[0], p[1], y1, dis, b1.reshape(1, HID), W2, HID)

    q = _sc_agg(ec, y2, HID)
    z = _tc_mid(q[0], q[1], y2, dis, b2.reshape(1, HID), W3, 1)

    r = _sc_agg(ec, z, 1)
    out = _tc_last(r[0], r[1], z, dis, b3.reshape(1, 1))
    return out[:N, 0]


# speedup vs baseline: 1.8934x; 1.0418x over previous
import jax
import jax.numpy as jnp
from jax import lax
from jax.experimental import pallas as pl
from jax.experimental.pallas import tpu as pltpu
from jax.experimental.pallas import tpu_sc as plsc

N = 10000
NP = 10240
E = 320000
IN_CH = 128
HID = 64

NC = 2
NS = 16
NW = NC * NS
K = 128
NBUF = 6
CH = 79
EP = NW * K * CH
RPT = NP // NS

RB = 1024
NB = NP // RB


def _sc_mesh():
    return plsc.VectorSubcoreMesh(
        core_axis_name="c", subcore_axis_name="s", num_cores=NC, num_subcores=NS
    )


def _sc_agg(ec, y, d):
    def body(src_h, dst_h, y_h, zz_h, out_h, sidx, didx, rows, acc, sem):
        c = lax.axis_index("c")
        s = lax.axis_index("s")
        wid = c * NS + s
        r0 = s * RPT
        pltpu.sync_copy(zz_h.at[pl.ds(r0, RPT)], acc.at[pl.ds(r0, RPT)])
        plsc.subcore_barrier()
        base = wid * (K * CH)

        def step(i, carry):
            off = base + i * K
            pltpu.sync_copy(src_h.at[pl.ds(off, K)], sidx)
            pltpu.sync_copy(dst_h.at[pl.ds(off, K)], didx)
            pltpu.async_copy(y_h.at[sidx], rows, sem).wait()
            pltpu.sync_copy(rows, acc.at[didx], add=True)
            return carry

        lax.fori_loop(0, CH, step, 0)
        plsc.subcore_barrier()
        pltpu.sync_copy(acc.at[pl.ds(r0, RPT)], out_h.at[c, pl.ds(r0, RPT)])

    zeros = jnp.zeros((NP, d), jnp.float32)
    f = pl.kernel(
        body,
        out_type=jax.ShapeDtypeStruct((NC, NP, d), jnp.float32),
        mesh=_sc_mesh(),
        compiler_params=pltpu.CompilerParams(use_tc_tiling_on_sc=False),
        scratch_types=[
            pltpu.VMEM((K,), jnp.int32),
            pltpu.VMEM((K,), jnp.int32),
            pltpu.VMEM((K, d), jnp.float32),
            pltpu.VMEM_SHARED((NP, d), jnp.float32),
            pltpu.SemaphoreType.DMA,
        ],
    )
    return f(ec[0], ec[1], y, zeros)


def _tc_first(d0, d1, x, W1):
    def body(d0_r, d1_r, x_r, w_r, dis_r, y_r):
        dis = lax.rsqrt(d0_r[...] + d1_r[...] + 1.0)
        xw = jnp.dot(x_r[...], w_r[...], preferred_element_type=jnp.float32)
        dis_r[...] = dis
        y_r[...] = dis * xw

    return pl.pallas_call(
        body,
        grid=(NB,),
        in_specs=[
            pl.BlockSpec((RB, 1), lambda i: (i, 0)),
            pl.BlockSpec((RB, 1), lambda i: (i, 0)),
            pl.BlockSpec((RB, IN_CH), lambda i: (i, 0)),
            pl.BlockSpec((IN_CH, HID), lambda i: (0, 0)),
        ],
        out_specs=[
            pl.BlockSpec((RB, 1), lambda i: (i, 0)),
            pl.BlockSpec((RB, HID), lambda i: (i, 0)),
        ],
        out_shape=[
            jax.ShapeDtypeStruct((NP, 1), jnp.float32),
            jax.ShapeDtypeStruct((NP, HID), jnp.float32),
        ],
    )(d0, d1, x, W1)


def _tc_mid(p0, p1, y, dis, b, W, d_out):
    def body(p0_r, p1_r, y_r, dis_r, b_r, w_r, o_r):
        dis = dis_r[...]
        h = jnp.maximum(dis * (p0_r[...] + p1_r[...] + y_r[...]) + b_r[...], 0.0)
        o_r[...] = dis * jnp.dot(h, w_r[...], preferred_element_type=jnp.float32)

    return pl.pallas_call(
        body,
        grid=(NB,),
        in_specs=[
            pl.BlockSpec((RB, HID), lambda i: (i, 0)),
            pl.BlockSpec((RB, HID), lambda i: (i, 0)),
            pl.BlockSpec((RB, HID), lambda i: (i, 0)),
            pl.BlockSpec((RB, 1), lambda i: (i, 0)),
            pl.BlockSpec((1, HID), lambda i: (0, 0)),
            pl.BlockSpec((HID, d_out), lambda i: (0, 0)),
        ],
        out_specs=pl.BlockSpec((RB, d_out), lambda i: (i, 0)),
        out_shape=jax.ShapeDtypeStruct((NP, d_out), jnp.float32),
    )(p0, p1, y, dis, b, W)


def _tc_last(r0, r1, z, dis, b3):
    def body(r0_r, r1_r, z_r, dis_r, b_r, o_r):
        o_r[...] = dis_r[...] * (r0_r[...] + r1_r[...] + z_r[...]) + b_r[...]

    return pl.pallas_call(
        body,
        grid=(NB,),
        in_specs=[
            pl.BlockSpec((RB, 1), lambda i: (i, 0)),
            pl.BlockSpec((RB, 1), lambda i: (i, 0)),
            pl.BlockSpec((RB, 1), lambda i: (i, 0)),
            pl.BlockSpec((RB, 1), lambda i: (i, 0)),
            pl.BlockSpec((1, 1), lambda i: (0, 0)),
        ],
        out_specs=pl.BlockSpec((RB, 1), lambda i: (i, 0)),
        out_shape=jax.ShapeDtypeStruct((NP, 1), jnp.float32),
    )(r0, r1, z, dis, b3)


def kernel(x, edge_index, W1, b1, W2, b2, W3, b3):
    ei = edge_index.astype(jnp.int32)
    pad = jnp.full((2, EP - E), N, jnp.int32)
    ec = jnp.concatenate([ei, pad], axis=1)
    xp = jnp.zeros((NP, IN_CH), jnp.float32).at[:N].set(x)

    DS = 16
    ones16 = jnp.ones((NP, DS), jnp.float32)
    degp = _sc_agg(ec, ones16, DS)
    dis, y1 = _tc_first(degp[0, :, 0:1], degp[1, :, 0:1], xp, W1)

    p = _sc_agg(ec, y1, HID)
    y2 = _tc_mid(p[0], p[1], y1, dis, b1.reshape(1, HID), W2, HID)

    q = _sc_agg(ec, y2, HID)
    z = _tc_mid(q[0], q[1], y2, dis, b2.reshape(1, HID), W3, 1)

    z16 = jnp.pad(z, ((0, 0), (0, DS - 1)))
    r = _sc_agg(ec, z16, DS)
    out = _tc_last(r[0, :, 0:1], r[1, :, 0:1], z, dis, b3.reshape(1, 1))
    return out[:N, 0]
